# baseline (speedup 1.0000x reference)
import jax
import jax.numpy as jnp
from jax.experimental import pallas as pl
from jax.experimental.pallas import tpu as pltpu
from jax.experimental.pallas import tpu_sc as plsc

_BLK_ROWS = 16
_LANES = 16


def kernel(x, pos_table):
    batch, seq_len, d_model = x.shape
    total_rows = batch * seq_len
    x2 = x.reshape(total_rows, d_model)
    pos = pos_table[:seq_len]
    n_blocks = total_rows // _BLK_ROWS
    n_pos_blocks = seq_len // _BLK_ROWS

    mesh = plsc.VectorSubcoreMesh(core_axis_name="core", subcore_axis_name="subcore")

    @pl.kernel(
        out_type=jax.ShapeDtypeStruct((total_rows, d_model), x.dtype),
        mesh=mesh,
    )
    def sc_add(x_hbm, pos_hbm, o_hbm):
        def body(x_vmem, pos_vmem, o_vmem):
            @pl.loop(0, _BLK_ROWS)
            def _(r):
                @pl.loop(0, d_model, step=_LANES * 8)
                def _(c):
                    for u in range(8):
                        slc = (pl.ds(r, 1), pl.ds(c + u * _LANES, _LANES))
                        o_vmem.at[*slc][...] = (
                            x_vmem.at[*slc][...] + pos_vmem.at[*slc][...]
                        )

        pltpu.emit_pipeline(
            body,
            grid=(n_blocks,),
            in_specs=[
                pl.BlockSpec((_BLK_ROWS, d_model), index_map=lambda i: (i, 0)),
                pl.BlockSpec(
                    (_BLK_ROWS, d_model),
                    index_map=lambda i: (i % n_pos_blocks, 0),
                ),
            ],
            out_specs=[
                pl.BlockSpec((_BLK_ROWS, d_model), index_map=lambda i: (i, 0))
            ],
            core_axis_name=("core", "subcore"),
            dimension_semantics=(pltpu.PARALLEL,),
        )(x_hbm, pos_hbm, o_hbm)

    return sc_add(x2, pos).reshape(batch, seq_len, d_model)


# speedup vs baseline: 1.3120x; 1.3120x over previous
import jax
import jax.numpy as jnp
from jax.experimental import pallas as pl
from jax.experimental.pallas import tpu as pltpu
from jax.experimental.pallas import tpu_sc as plsc

_BS = 2048
_BLK_ROWS = 16
_LANES = 16


def _add_kernel(x_ref, pos_ref, out_ref):
    out_ref[...] = x_ref[...] + pos_ref[...]


def kernel(x, pos_table):
    batch, seq_len, d_model = x.shape
    pos = pos_table[:seq_len]
    total_rows = batch * seq_len
    x2 = x.reshape(total_rows, d_model)

    tc_rows = 13 * _BS
    sc_rows = total_rows - tc_rows
    n_pos_blocks_tc = seq_len // _BS

    tc_part = pl.pallas_call(
        _add_kernel,
        grid=(tc_rows // _BS,),
        in_specs=[
            pl.BlockSpec((_BS, d_model), lambda i: (i, 0)),
            pl.BlockSpec((_BS, d_model), lambda i: (i % n_pos_blocks_tc, 0)),
        ],
        out_specs=pl.BlockSpec((_BS, d_model), lambda i: (i, 0)),
        out_shape=jax.ShapeDtypeStruct((tc_rows, d_model), x.dtype),
        compiler_params=pltpu.CompilerParams(
            dimension_semantics=("parallel",),
        ),
    )(x2[:tc_rows], pos)

    mesh = plsc.VectorSubcoreMesh(core_axis_name="core", subcore_axis_name="subcore")
    n_sc_blocks = sc_rows // _BLK_ROWS
    n_pos_blocks_sc = seq_len // _BLK_ROWS
    sc_block_off = tc_rows // _BLK_ROWS

    @pl.kernel(
        out_type=jax.ShapeDtypeStruct((sc_rows, d_model), x.dtype),
        mesh=mesh,
    )
    def sc_add(x_hbm, pos_hbm, o_hbm):
        def body(x_vmem, pos_vmem, o_vmem):
            @pl.loop(0, _BLK_ROWS)
            def _(r):
                @pl.loop(0, d_model, step=_LANES * 4)
                def _(c):
                    for u in range(4):
                        slc = (pl.ds(r, 1), pl.ds(c + u * _LANES, _LANES))
                        o_vmem.at[*slc][...] = (
                            x_vmem.at[*slc][...] + pos_vmem.at[*slc][...]
                        )

        pltpu.emit_pipeline(
            body,
            grid=(n_sc_blocks,),
            in_specs=[
                pl.BlockSpec((_BLK_ROWS, d_model), index_map=lambda i: (i, 0)),
                pl.BlockSpec(
                    (_BLK_ROWS, d_model),
                    index_map=lambda i: ((sc_block_off + i) % n_pos_blocks_sc, 0),
                ),
            ],
            out_specs=[
                pl.BlockSpec((_BLK_ROWS, d_model), index_map=lambda i: (i, 0))
            ],
            core_axis_name=("core", "subcore"),
            dimension_semantics=(pltpu.PARALLEL,),
        )(x_hbm, pos_hbm, o_hbm)

    sc_part = sc_add(x2[tc_rows:], pos)

    out2 = jnp.concatenate([tc_part, sc_part], axis=0)
    return out2.reshape(batch, seq_len, d_model)


# speedup vs baseline: 4.3760x; 3.3355x over previous
import jax
import jax.numpy as jnp
from jax.experimental import pallas as pl
from jax.experimental.pallas import tpu as pltpu

_BS = 2048


def _add_kernel(x_ref, pos_ref, out_ref):
    out_ref[...] = x_ref[...] + pos_ref[...]


def kernel(x, pos_table):
    batch, seq_len, d_model = x.shape
    pos = pos_table[:seq_len]
    ns = seq_len // _BS
    return pl.pallas_call(
        _add_kernel,
        grid=(ns, batch),
        in_specs=[
            pl.BlockSpec((1, _BS, d_model), lambda s, b: (b, s, 0)),
            pl.BlockSpec((_BS, d_model), lambda s, b: (s, 0)),
        ],
        out_specs=pl.BlockSpec((1, _BS, d_model), lambda s, b: (b, s, 0)),
        out_shape=jax.ShapeDtypeStruct((batch, seq_len, d_model), x.dtype),
        compiler_params=pltpu.CompilerParams(
            dimension_semantics=("parallel", "parallel"),
        ),
    )(x, pos)
